# baseline (speedup 1.0000x reference)
import functools

import jax
import jax.numpy as jnp
from jax import lax
from jax.experimental import pallas as pl
from jax.experimental.pallas import tpu as pltpu
from jax.experimental.pallas import tpu_sc as plsc

_N = 262144
_R = 4096
_D = 32
_SB = 256
_NSUB = 64
_B = _SB * _NSUB
_NB = _N // _B

_SC_SUB = 16
_SC_CHUNK = _N // _SC_SUB
_SC_T = 1024
_SC_NT = _SC_CHUNK // _SC_T
_R_SLICE = _R // _SC_SUB


def _seg_psum(ids, v):
    pos_c = jax.lax.broadcasted_iota(jnp.int32, (_SB, 1), 0)
    pos_r = jax.lax.broadcasted_iota(jnp.int32, (1, _SB), 1)
    kc = ids.reshape(_SB, 1) * _SB + pos_c
    kr = ids.reshape(1, _SB) * _SB + pos_r
    maskf = ((kc - kr).astype(jnp.uint32) < _SB).astype(jnp.float32)
    return jax.lax.dot_general(
        maskf, v, (((1,), (0,)), ((), ())), preferred_element_type=jnp.float32
    )


_VSPLIT = 4
_BSPL = _B // _VSPLIT
_SUB_PER_SPL = _NSUB // _VSPLIT


def _scan_body(*refs):
    v_refs = refs[:_VSPLIT]
    ids_ref = refs[_VSPLIT]
    out_ref = refs[_VSPLIT + 1]
    carry_ref = refs[_VSPLIT + 2]
    cid_ref = refs[_VSPLIT + 3]
    b = pl.program_id(0)

    @pl.when(b == 0)
    def _init():
        carry_ref[...] = jnp.zeros((1, _D), jnp.float32)
        cid_ref[0] = jnp.int32(-1)

    ids = ids_ref[0, 0, :]
    prev_id = cid_ref[0]
    carry = carry_ref[...]

    off = 0
    for s in range(_NSUB):
        ids_s = lax.slice(ids, (off,), (off + _SB,))
        v_s = v_refs[s // _SUB_PER_SPL][
            pl.ds((s % _SUB_PER_SPL) * _SB, _SB), :
        ]
        ps = _seg_psum(ids_s, v_s)
        match = (ids_s == prev_id).astype(jnp.float32).reshape(_SB, 1)
        out = ps + match * carry
        out_ref[pl.ds(off, _SB), :] = out
        carry = out[_SB - 1 :, :]
        prev_id = ids_s[_SB - 1]
        off += _SB

    carry_ref[...] = carry
    cid_ref[0] = prev_id


def _per_sample_call(sample_values, ids3):
    n, d = sample_values.shape
    nb = n // _B
    v_specs = [
        pl.BlockSpec((_BSPL, d), functools.partial(
            lambda k, i: (i * _VSPLIT + k, 0), k))
        for k in range(_VSPLIT)
    ]
    return pl.pallas_call(
        _scan_body,
        grid=(nb,),
        in_specs=v_specs + [
            pl.BlockSpec((1, 1, _B), lambda i: (i, 0, 0)),
        ],
        out_specs=pl.BlockSpec((_B, d), lambda i: (i, 0)),
        out_shape=jax.ShapeDtypeStruct((n, d), jnp.float32),
        scratch_shapes=[
            pltpu.VMEM((1, _D), jnp.float32),
            pltpu.SMEM((1,), jnp.int32),
        ],
        compiler_params=pltpu.CompilerParams(
            dimension_semantics=("arbitrary",)
        ),
    )(*([sample_values] * _VSPLIT), ids3)


def _sc_perray_body(v_hbm, ids_hbm, out_hbm, acc, zbuf, rows, idx):
    cid = lax.axis_index("c")
    sid = lax.axis_index("s")

    @pl.when(cid == 0)
    def _core0():
        def _zb(i, c):
            zbuf[i, pl.ds(0, 16)] = jnp.zeros((16,), jnp.float32)
            zbuf[i, pl.ds(16, 16)] = jnp.zeros((16,), jnp.float32)
            return c

        lax.fori_loop(0, _R_SLICE, _zb, 0)
        racc0 = pl.multiple_of(sid * _R_SLICE, 8)
        pltpu.sync_copy(zbuf, acc.at[pl.ds(racc0, _R_SLICE)])
        plsc.subcore_barrier()

        def _tile(t, c):
            off = pl.multiple_of(sid * _SC_CHUNK + t * _SC_T, 8)
            pltpu.sync_copy(v_hbm.at[pl.ds(off, _SC_T)], rows)
            ioff = pl.multiple_of(
                sid * (_SC_CHUNK // 128) + t * (_SC_T // 128), 8
            )
            pltpu.sync_copy(ids_hbm.at[pl.ds(ioff, _SC_T // 128)], idx)
            for j in range(_SC_T // 128):
                pltpu.sync_copy(
                    rows.at[pl.ds(j * 128, 128)], acc.at[idx.at[j]], add=True
                )
            return c

        lax.fori_loop(0, _SC_NT, _tile, 0)

        plsc.subcore_barrier()
        pltpu.sync_copy(
            acc.at[pl.ds(sid * _R_SLICE, _R_SLICE)],
            out_hbm.at[pl.ds(sid * _R_SLICE, _R_SLICE)],
        )


@functools.partial(
    pl.kernel,
    out_type=jax.ShapeDtypeStruct((_R, _D), jnp.float32),
    mesh=plsc.VectorSubcoreMesh(core_axis_name="c", subcore_axis_name="s"),
    compiler_params=pltpu.CompilerParams(use_tc_tiling_on_sc=False),
    scratch_types=[
        pltpu.VMEM_SHARED((_R, _D), jnp.float32),
        pltpu.VMEM((_R_SLICE, _D), jnp.float32),
        pltpu.VMEM((_SC_T, _D), jnp.float32),
        pltpu.VMEM((_SC_T // 128, 128), jnp.int32),
    ],
)
def _per_ray_call(v_hbm, ids_hbm, out_hbm, acc, zbuf, rows, idx):
    _sc_perray_body(v_hbm, ids_hbm, out_hbm, acc, zbuf, rows, idx)


@jax.jit
def kernel(sample_values, ray_ids):
    n, d = sample_values.shape
    nb = n // _B
    ids32 = ray_ids.astype(jnp.int32)
    ids3 = ids32.reshape(nb, 1, _B)
    ids2d = ids32.reshape(n // 128, 128)

    out_ray = _per_ray_call(sample_values, ids2d)
    out_sample = _per_sample_call(sample_values, ids3)
    return out_ray, out_sample


# speedup vs baseline: 1.0086x; 1.0086x over previous
import functools

import jax
import jax.numpy as jnp
from jax import lax
from jax.experimental import pallas as pl
from jax.experimental.pallas import tpu as pltpu
from jax.experimental.pallas import tpu_sc as plsc

_N = 262144
_R = 4096
_D = 32
_SB = 256
_NSUB = 32
_B = _SB * _NSUB
_NB = _N // _B

_SC_SUB = 16
_SC_CHUNK = _N // _SC_SUB
_SC_T = 1024
_SC_NT = _SC_CHUNK // _SC_T
_R_SLICE = _R // _SC_SUB


def _seg_psum(ids, v):
    pos_c = jax.lax.broadcasted_iota(jnp.int32, (_SB, 1), 0)
    pos_r = jax.lax.broadcasted_iota(jnp.int32, (1, _SB), 1)
    kc = ids.reshape(_SB, 1) * _SB + pos_c
    kr = ids.reshape(1, _SB) * _SB + pos_r
    maskf = ((kc - kr).astype(jnp.uint32) < _SB).astype(jnp.float32)
    return jax.lax.dot_general(
        maskf, v, (((1,), (0,)), ((), ())), preferred_element_type=jnp.float32
    )


def _scan_body(v_ref, ids_ref, out_ref, carry_ref, cid_ref):
    b = pl.program_id(0)

    @pl.when(b == 0)
    def _init():
        carry_ref[...] = jnp.zeros((1, _D), jnp.float32)
        cid_ref[0] = jnp.int32(-1)

    ids = ids_ref[0, 0, :]
    prev_id = cid_ref[0]
    carry = carry_ref[...]

    off = 0
    for _ in range(_NSUB):
        ids_s = lax.slice(ids, (off,), (off + _SB,))
        v_s = v_ref[pl.ds(off, _SB), :]
        ps = _seg_psum(ids_s, v_s)
        match = (ids_s == prev_id).astype(jnp.float32).reshape(_SB, 1)
        out = ps + match * carry
        out_ref[pl.ds(off, _SB), :] = out
        carry = out[_SB - 1 :, :]
        prev_id = ids_s[_SB - 1]
        off += _SB

    carry_ref[...] = carry
    cid_ref[0] = prev_id


def _per_sample_call(sample_values, ids3):
    n, d = sample_values.shape
    nb = n // _B
    return pl.pallas_call(
        _scan_body,
        grid=(nb,),
        in_specs=[
            pl.BlockSpec((_B, d), lambda i: (i, 0)),
            pl.BlockSpec((1, 1, _B), lambda i: (i, 0, 0)),
        ],
        out_specs=pl.BlockSpec((_B, d), lambda i: (i, 0)),
        out_shape=jax.ShapeDtypeStruct((n, d), jnp.float32),
        scratch_shapes=[
            pltpu.VMEM((1, _D), jnp.float32),
            pltpu.SMEM((1,), jnp.int32),
        ],
        compiler_params=pltpu.CompilerParams(
            dimension_semantics=("arbitrary",)
        ),
    )(sample_values, ids3)


def _sc_perray_body(v_hbm, ids_hbm, out_hbm, acc, zbuf, rows, idx):
    cid = lax.axis_index("c")
    sid = lax.axis_index("s")

    @pl.when(cid == 0)
    def _core0():
        def _zb(i, c):
            zbuf[i, pl.ds(0, 16)] = jnp.zeros((16,), jnp.float32)
            zbuf[i, pl.ds(16, 16)] = jnp.zeros((16,), jnp.float32)
            return c

        lax.fori_loop(0, _R_SLICE, _zb, 0)
        racc0 = pl.multiple_of(sid * _R_SLICE, 8)
        pltpu.sync_copy(zbuf, acc.at[pl.ds(racc0, _R_SLICE)])
        plsc.subcore_barrier()

        def _tile(t, c):
            off = pl.multiple_of(sid * _SC_CHUNK + t * _SC_T, 8)
            pltpu.sync_copy(v_hbm.at[pl.ds(off, _SC_T)], rows)
            ioff = pl.multiple_of(
                sid * (_SC_CHUNK // 128) + t * (_SC_T // 128), 8
            )
            pltpu.sync_copy(ids_hbm.at[pl.ds(ioff, _SC_T // 128)], idx)
            for j in range(_SC_T // 128):
                pltpu.sync_copy(
                    rows.at[pl.ds(j * 128, 128)], acc.at[idx.at[j]], add=True
                )
            return c

        lax.fori_loop(0, _SC_NT, _tile, 0)

        plsc.subcore_barrier()
        pltpu.sync_copy(
            acc.at[pl.ds(sid * _R_SLICE, _R_SLICE)],
            out_hbm.at[pl.ds(sid * _R_SLICE, _R_SLICE)],
        )


@functools.partial(
    pl.kernel,
    out_type=jax.ShapeDtypeStruct((_R, _D), jnp.float32),
    mesh=plsc.VectorSubcoreMesh(core_axis_name="c", subcore_axis_name="s"),
    compiler_params=pltpu.CompilerParams(use_tc_tiling_on_sc=False),
    scratch_types=[
        pltpu.VMEM_SHARED((_R, _D), jnp.float32),
        pltpu.VMEM((_R_SLICE, _D), jnp.float32),
        pltpu.VMEM((_SC_T, _D), jnp.float32),
        pltpu.VMEM((_SC_T // 128, 128), jnp.int32),
    ],
)
def _per_ray_call(v_hbm, ids_hbm, out_hbm, acc, zbuf, rows, idx):
    _sc_perray_body(v_hbm, ids_hbm, out_hbm, acc, zbuf, rows, idx)


@jax.jit
def kernel(sample_values, ray_ids):
    n, d = sample_values.shape
    nb = n // _B
    ids32 = ray_ids.astype(jnp.int32)
    ids3 = ids32.reshape(nb, 1, _B)
    ids2d = ids32.reshape(n // 128, 128)

    out_ray = _per_ray_call(sample_values, ids2d)
    out_sample = _per_sample_call(sample_values, ids3)
    return out_ray, out_sample
